# baseline (speedup 1.0000x reference)
import functools

import jax
import jax.numpy as jnp
from jax.experimental import pallas as pl
from jax.experimental.pallas import tpu as pltpu

N = 10000
E = 160000
D = 128
FS = 128
EPS = 1e-5
EB = 4000
NSTEP = E // EB


def _dotT(a, b):
    return jax.lax.dot_general(a, b, (((1,), (1,)), ((), ())),
                               preferred_element_type=jnp.float32)


def _gf_body(x2_ref, w_ref, b_ref, o_ref):
    o_ref[...] = _dotT(x2_ref[...], w_ref[...]) + b_ref[...]


def _stats1_body(u_ref, v_ref, o_ref, acc_ref):
    i = pl.program_id(0)

    @pl.when(i == 0)
    def _():
        acc_ref[...] = jnp.zeros_like(acc_ref)

    u = u_ref[...]
    d = v_ref[...] - u
    acc_ref[0:1, :] = acc_ref[0:1, :] + jnp.sum(u, axis=0, keepdims=True)
    acc_ref[1:2, :] = acc_ref[1:2, :] + jnp.sum(u * u, axis=0, keepdims=True)
    acc_ref[2:3, :] = acc_ref[2:3, :] + jnp.sum(d, axis=0, keepdims=True)
    acc_ref[3:4, :] = acc_ref[3:4, :] + jnp.sum(d * d, axis=0, keepdims=True)

    @pl.when(i == pl.num_programs(0) - 1)
    def _():
        o_ref[...] = acc_ref[...]


def _mm1_body(s1_ref, g1_ref, b1_ref, w1_ref, u_ref, v_ref, h1_ref, o2_ref,
              acc_ref):
    i = pl.program_id(0)
    inv_e = 1.0 / E
    mu = s1_ref[0:1, :] * inv_e
    vu = s1_ref[1:2, :] * inv_e - mu * mu
    md = s1_ref[2:3, :] * inv_e
    vd = s1_ref[3:4, :] * inv_e - md * md
    sa = g1_ref[0:1, :] * jax.lax.rsqrt(vu + EPS)
    sb = g1_ref[1:2, :] * jax.lax.rsqrt(vd + EPS)
    ba = b1_ref[0:1, :]
    bb = b1_ref[1:2, :]

    u = u_ref[...]
    d = v_ref[...] - u
    pa = jnp.maximum((u - mu) * sa + ba, 0.0)
    pb = jnp.maximum((d - md) * sb + bb, 0.0)
    w = w1_ref[...]
    h = _dotT(pa, w[:, :FS]) + _dotT(pb, w[:, FS:])
    h1_ref[...] = h

    @pl.when(i == 0)
    def _():
        acc_ref[...] = jnp.zeros_like(acc_ref)

    acc_ref[0:1, :] = acc_ref[0:1, :] + jnp.sum(h, axis=0, keepdims=True)
    acc_ref[1:2, :] = acc_ref[1:2, :] + jnp.sum(h * h, axis=0, keepdims=True)

    @pl.when(i == pl.num_programs(0) - 1)
    def _():
        o2_ref[...] = acc_ref[...]


def _mm2_body(s2_ref, g2_ref, b2_ref, w2_ref, h1_ref, h2_ref):
    inv_e = 1.0 / E
    m = s2_ref[0:1, :] * inv_e
    v = s2_ref[1:2, :] * inv_e - m * m
    sc = g2_ref[...] * jax.lax.rsqrt(v + EPS)
    p = jnp.maximum((h1_ref[...] - m) * sc + b2_ref[...], 0.0)
    h2_ref[...] = _dotT(p, w2_ref[...])


def _out_body(g_ref, w_ref, b_ref, o_ref):
    o_ref[...] = _dotT(g_ref[...], w_ref[...]) + b_ref[...]


_full = lambda shp: pl.BlockSpec(shp, lambda i: tuple(0 for _ in shp))


def _gf_call(x2, wav, bav):
    nb = 2000
    return pl.pallas_call(
        _gf_body,
        grid=(N // nb,),
        in_specs=[
            pl.BlockSpec((nb, 2 * D), lambda i: (i, 0)),
            _full((FS, 2 * D)),
            _full((1, FS)),
        ],
        out_specs=pl.BlockSpec((nb, FS), lambda i: (i, 0)),
        out_shape=jax.ShapeDtypeStruct((N, FS), jnp.float32),
    )(x2, wav, bav)


def _stats1_call(u, v):
    return pl.pallas_call(
        _stats1_body,
        grid=(NSTEP,),
        in_specs=[
            pl.BlockSpec((EB, FS), lambda i: (i, 0)),
            pl.BlockSpec((EB, FS), lambda i: (i, 0)),
        ],
        out_specs=_full((8, FS)),
        out_shape=jax.ShapeDtypeStruct((8, FS), jnp.float32),
        scratch_shapes=[pltpu.VMEM((8, FS), jnp.float32)],
    )(u, v)


def _mm1_call(s1, g1, b1, w1, u, v):
    return pl.pallas_call(
        _mm1_body,
        grid=(NSTEP,),
        in_specs=[
            _full((8, FS)),
            _full((2, FS)),
            _full((2, FS)),
            _full((FS, 2 * FS)),
            pl.BlockSpec((EB, FS), lambda i: (i, 0)),
            pl.BlockSpec((EB, FS), lambda i: (i, 0)),
        ],
        out_specs=[
            pl.BlockSpec((EB, FS), lambda i: (i, 0)),
            _full((8, FS)),
        ],
        out_shape=[
            jax.ShapeDtypeStruct((E, FS), jnp.float32),
            jax.ShapeDtypeStruct((8, FS), jnp.float32),
        ],
        scratch_shapes=[pltpu.VMEM((8, FS), jnp.float32)],
    )(s1, g1, b1, w1, u, v)


def _mm2_call(s2, g2, b2, w2, h1):
    return pl.pallas_call(
        _mm2_body,
        grid=(NSTEP,),
        in_specs=[
            _full((8, FS)),
            _full((1, FS)),
            _full((1, FS)),
            _full((FS, FS)),
            pl.BlockSpec((EB, FS), lambda i: (i, 0)),
        ],
        out_specs=pl.BlockSpec((EB, FS), lambda i: (i, 0)),
        out_shape=jax.ShapeDtypeStruct((E, FS), jnp.float32),
    )(s2, g2, b2, w2, h1)


def _out_call(g4, wout_pad, bout_pad):
    nb = 2000
    return pl.pallas_call(
        _out_body,
        grid=(N // nb,),
        in_specs=[
            pl.BlockSpec((nb, FS), lambda i: (i, 0)),
            _full((FS, FS)),
            _full((1, FS)),
        ],
        out_specs=pl.BlockSpec((nb, FS), lambda i: (i, 0)),
        out_shape=jax.ShapeDtypeStruct((N, FS), jnp.float32),
    )(g4, wout_pad, bout_pad)


def _edge_conv(g, src, dst, g1, b1, w1, g2, b2, w2):
    u = jnp.take(g, dst, axis=0)
    v = jnp.take(g, src, axis=0)
    s1 = _stats1_call(u, v)
    h1, s2 = _mm1_call(s1, g1.reshape(2, FS), b1.reshape(2, FS), w1, u, v)
    h2 = _mm2_call(s2, g2.reshape(1, FS), b2.reshape(1, FS), w2, h1)
    seg = jax.ops.segment_max(h2, dst, num_segments=N)
    return jnp.where(jnp.isfinite(seg), seg, 0.0)


def kernel(x, edge_index, Wav, bav, l1_g1, l1_b1, l1_W1, l1_g2, l1_b2, l1_W2,
           l2_g1, l2_b1, l2_W1, l2_g2, l2_b2, l2_W2, l3_g1, l3_b1, l3_W1,
           l3_g2, l3_b2, l3_W2, l4_g1, l4_b1, l4_W1, l4_g2, l4_b2, l4_W2,
           Wout, bout):
    src = edge_index[0]
    dst = edge_index[1]
    x2 = x.reshape(N, 2 * D)
    gf = _gf_call(x2, Wav, bav.reshape(1, FS))
    p1 = (l1_g1, l1_b1, l1_W1, l1_g2, l1_b2, l1_W2)
    p2 = (l2_g1, l2_b1, l2_W1, l2_g2, l2_b2, l2_W2)
    p3 = (l3_g1, l3_b1, l3_W1, l3_g2, l3_b2, l3_W2)
    p4 = (l4_g1, l4_b1, l4_W1, l4_g2, l4_b2, l4_W2)
    g1 = _edge_conv(gf, src, dst, *p1)
    g2 = _edge_conv(g1, src, dst, *p2) + g1
    g3 = _edge_conv(g2, src, dst, *p3) + g2
    g4 = _edge_conv(g3, src, dst, *p4) + g3
    wout_pad = jnp.zeros((FS, FS), jnp.float32).at[:2, :].set(Wout)
    bout_pad = jnp.zeros((1, FS), jnp.float32).at[0, :2].set(bout)
    out_pad = _out_call(g4, wout_pad, bout_pad)
    return out_pad[:, :2]


# speedup vs baseline: 1.6193x; 1.6193x over previous
import functools

import jax
import jax.numpy as jnp
from jax import lax
from jax.experimental import pallas as pl
from jax.experimental.pallas import tpu as pltpu
from jax.experimental.pallas import tpu_sc as plsc

N = 10000
E = 160000
D = 128
FS = 128
EPS = 1e-5
EB = 4000
NSTEP = E // EB

SC_NC = 2
SC_NS = 16
SC_NW = SC_NC * SC_NS
NPB = 313
NPAD = SC_NW * NPB
FLUSH = 2048
CAPR = E + FLUSH
BCHUNK = 2000
GSEG = 128
TRASH = 511
ACCROWS = 512
NEG_INIT = float(jnp.finfo(jnp.float32).min)


def _dotT(a, b):
    return jax.lax.dot_general(a, b, (((1,), (1,)), ((), ())),
                               preferred_element_type=jnp.float32)


def _gf_body(x2_ref, w_ref, b_ref, o_ref):
    o_ref[...] = _dotT(x2_ref[...], w_ref[...]) + b_ref[...]


def _stats1_body(u_ref, v_ref, o_ref, acc_ref):
    i = pl.program_id(0)

    @pl.when(i == 0)
    def _():
        acc_ref[...] = jnp.zeros_like(acc_ref)

    u = u_ref[...]
    d = v_ref[...] - u
    acc_ref[0:1, :] = acc_ref[0:1, :] + jnp.sum(u, axis=0, keepdims=True)
    acc_ref[1:2, :] = acc_ref[1:2, :] + jnp.sum(u * u, axis=0, keepdims=True)
    acc_ref[2:3, :] = acc_ref[2:3, :] + jnp.sum(d, axis=0, keepdims=True)
    acc_ref[3:4, :] = acc_ref[3:4, :] + jnp.sum(d * d, axis=0, keepdims=True)

    @pl.when(i == pl.num_programs(0) - 1)
    def _():
        o_ref[...] = acc_ref[...]


def _mm1_body(s1_ref, g1_ref, b1_ref, w1_ref, u_ref, v_ref, h1_ref, o2_ref,
              acc_ref):
    i = pl.program_id(0)
    inv_e = 1.0 / E
    mu = s1_ref[0:1, :] * inv_e
    vu = s1_ref[1:2, :] * inv_e - mu * mu
    md = s1_ref[2:3, :] * inv_e
    vd = s1_ref[3:4, :] * inv_e - md * md
    sa = g1_ref[0:1, :] * jax.lax.rsqrt(vu + EPS)
    sb = g1_ref[1:2, :] * jax.lax.rsqrt(vd + EPS)
    ba = b1_ref[0:1, :]
    bb = b1_ref[1:2, :]

    u = u_ref[...]
    d = v_ref[...] - u
    pa = jnp.maximum((u - mu) * sa + ba, 0.0)
    pb = jnp.maximum((d - md) * sb + bb, 0.0)
    w = w1_ref[...]
    h = _dotT(pa, w[:, :FS]) + _dotT(pb, w[:, FS:])
    h1_ref[...] = h

    @pl.when(i == 0)
    def _():
        acc_ref[...] = jnp.zeros_like(acc_ref)

    acc_ref[0:1, :] = acc_ref[0:1, :] + jnp.sum(h, axis=0, keepdims=True)
    acc_ref[1:2, :] = acc_ref[1:2, :] + jnp.sum(h * h, axis=0, keepdims=True)

    @pl.when(i == pl.num_programs(0) - 1)
    def _():
        o2_ref[...] = acc_ref[...]


def _mm2_body(s2_ref, g2_ref, b2_ref, w2_ref, h1_ref, h2_ref):
    inv_e = 1.0 / E
    m = s2_ref[0:1, :] * inv_e
    v = s2_ref[1:2, :] * inv_e - m * m
    sc = g2_ref[...] * jax.lax.rsqrt(v + EPS)
    p = jnp.maximum((h1_ref[...] - m) * sc + b2_ref[...], 0.0)
    h2_ref[...] = _dotT(p, w2_ref[...])


def _out_body(g_ref, w_ref, b_ref, o_ref):
    o_ref[...] = _dotT(g_ref[...], w_ref[...]) + b_ref[...]


_full = lambda shp: pl.BlockSpec(shp, lambda i: tuple(0 for _ in shp))


def _gf_call(x2, wav, bav):
    nb = 2000
    return pl.pallas_call(
        _gf_body,
        grid=(N // nb,),
        in_specs=[
            pl.BlockSpec((nb, 2 * D), lambda i: (i, 0)),
            _full((FS, 2 * D)),
            _full((1, FS)),
        ],
        out_specs=pl.BlockSpec((nb, FS), lambda i: (i, 0)),
        out_shape=jax.ShapeDtypeStruct((N, FS), jnp.float32),
    )(x2, wav, bav)


def _stats1_call(u, v):
    return pl.pallas_call(
        _stats1_body,
        grid=(NSTEP,),
        in_specs=[
            pl.BlockSpec((EB, FS), lambda i: (i, 0)),
            pl.BlockSpec((EB, FS), lambda i: (i, 0)),
        ],
        out_specs=_full((8, FS)),
        out_shape=jax.ShapeDtypeStruct((8, FS), jnp.float32),
        scratch_shapes=[pltpu.VMEM((8, FS), jnp.float32)],
    )(u, v)


def _mm1_call(s1, g1, b1, w1, u, v):
    return pl.pallas_call(
        _mm1_body,
        grid=(NSTEP,),
        in_specs=[
            _full((8, FS)),
            _full((2, FS)),
            _full((2, FS)),
            _full((FS, 2 * FS)),
            pl.BlockSpec((EB, FS), lambda i: (i, 0)),
            pl.BlockSpec((EB, FS), lambda i: (i, 0)),
        ],
        out_specs=[
            pl.BlockSpec((EB, FS), lambda i: (i, 0)),
            _full((8, FS)),
        ],
        out_shape=[
            jax.ShapeDtypeStruct((E, FS), jnp.float32),
            jax.ShapeDtypeStruct((8, FS), jnp.float32),
        ],
        scratch_shapes=[pltpu.VMEM((8, FS), jnp.float32)],
    )(s1, g1, b1, w1, u, v)


def _mm2_call(s2, g2, b2, w2, h1):
    return pl.pallas_call(
        _mm2_body,
        grid=(NSTEP,),
        in_specs=[
            _full((8, FS)),
            _full((1, FS)),
            _full((1, FS)),
            _full((FS, FS)),
            pl.BlockSpec((EB, FS), lambda i: (i, 0)),
        ],
        out_specs=pl.BlockSpec((EB, FS), lambda i: (i, 0)),
        out_shape=jax.ShapeDtypeStruct((E, FS), jnp.float32),
    )(s2, g2, b2, w2, h1)


def _out_call(g4, wout_pad, bout_pad):
    nb = 2000
    return pl.pallas_call(
        _out_body,
        grid=(N // nb,),
        in_specs=[
            pl.BlockSpec((nb, FS), lambda i: (i, 0)),
            _full((FS, FS)),
            _full((1, FS)),
        ],
        out_specs=pl.BlockSpec((nb, FS), lambda i: (i, 0)),
        out_shape=jax.ShapeDtypeStruct((N, FS), jnp.float32),
    )(g4, wout_pad, bout_pad)


def _sc_mesh():
    return plsc.VectorSubcoreMesh(core_axis_name="c", subcore_axis_name="s",
                                  num_cores=SC_NC, num_subcores=SC_NS)


def _wid():
    return lax.axis_index("s") * SC_NC + lax.axis_index("c")


def _bucket_body(dst_hbm, ptab_hbm, cnt_hbm, dstc_v, pbuf_v, csplat_v):
    w = _wid()
    lo = w * NPB
    hi = lo + NPB
    iota = lax.iota(jnp.int32, 16)

    def init_body(i, _):
        pbuf_v[pl.ds(i * 16, 16)] = jnp.full((16,), TRASH, jnp.int32)
        return 0

    lax.fori_loop(0, (FLUSH + 16) // 16, init_body, 0)

    def chunk_body(j, carry):
        pltpu.sync_copy(dst_hbm.at[pl.ds(j * BCHUNK, BCHUNK)], dstc_v)

        def vreg_body(i, c2):
            n, written = c2
            d16 = dstc_v[pl.ds(i * 16, 16)]
            ebase = (j * BCHUNK + i * 16) * 512
            mks = []
            for k in range(16):
                d = d16[k]
                mks.append(((d >= lo) & (d < hi)).astype(jnp.int32))
            any_m = mks[0]
            for k in range(1, 16):
                any_m = any_m | mks[k]

            @pl.when(any_m > 0)
            def _():
                nn = n
                for k in range(16):
                    d = d16[k]
                    mk = mks[k] > 0

                    @pl.when(mk)
                    def _():
                        val = ebase + k * 512 + (d - lo)
                        slot = (nn >> 4) << 4
                        off = nn & 15
                        cur = pbuf_v[pl.ds(slot, 16)]
                        pbuf_v[pl.ds(slot, 16)] = jnp.where(
                            iota == off, val, cur)

                    nn = nn + mks[k]

            for k in range(16):
                n = n + mks[k]
            do_flush = n >= FLUSH

            @pl.when(do_flush)
            def _():
                woff = w * CAPR + pl.multiple_of(written, 8)
                pltpu.sync_copy(pbuf_v.at[pl.ds(0, FLUSH)],
                                ptab_hbm.at[pl.ds(woff, FLUSH)])
                pbuf_v[pl.ds(0, 16)] = pbuf_v[pl.ds(FLUSH, 16)]
                pbuf_v[pl.ds(FLUSH, 16)] = jnp.full((16,), TRASH, jnp.int32)

            n = jnp.where(do_flush, n - FLUSH, n)
            written = jnp.where(do_flush, written + FLUSH, written)
            return (n, written)

        return lax.fori_loop(0, BCHUNK // 16, vreg_body, carry)

    n, written = lax.fori_loop(0, E // BCHUNK, chunk_body,
                               (jnp.int32(0), jnp.int32(0)))
    woff = w * CAPR + pl.multiple_of(written, 8)
    pltpu.sync_copy(pbuf_v.at[pl.ds(0, FLUSH)],
                    ptab_hbm.at[pl.ds(woff, FLUSH)])
    csplat_v[...] = jnp.broadcast_to(written + n, (16,))
    pltpu.sync_copy(csplat_v, cnt_hbm.at[pl.ds(w * 16, 16)])


def _bucket_call(dst):
    f = pl.kernel(
        _bucket_body,
        out_type=[
            jax.ShapeDtypeStruct((SC_NW * CAPR,), jnp.int32),
            jax.ShapeDtypeStruct((SC_NW * 16,), jnp.int32),
        ],
        mesh=_sc_mesh(),
        scratch_types=[
            pltpu.VMEM((BCHUNK,), jnp.int32),
            pltpu.VMEM((FLUSH + 16,), jnp.int32),
            pltpu.VMEM((16,), jnp.int32),
        ],
    )
    return f(dst)


GSEG2 = 64


def _segmax_body(h2_hbm, ptab_hbm, cnt_hbm, out_hbm,
                 pkA, pkB, idxA, idxB, rowsA, rowsB, acc_v, cnt_v,
                 semA, semB):
    w = _wid()
    lo = w * NPB

    def init_body(i, _):
        acc_v[pl.ds(i * 16, 16)] = jnp.full((16,), NEG_INIT, jnp.float32)
        return 0

    lax.fori_loop(0, ACCROWS * FS // 16, init_body, 0)

    pltpu.sync_copy(cnt_hbm.at[pl.ds(w * 16, 16)], cnt_v)
    count = cnt_v[...][0]
    nch = (count + GSEG2 - 1) >> 6

    def prep_fire(c, pk, idx, rows, sem):
        base = w * CAPR + c * GSEG2
        pltpu.sync_copy(ptab_hbm.at[pl.ds(base, GSEG2)], pk)
        for g in range(GSEG2 // 16):
            idx[pl.ds(g * 16, 16)] = pk[pl.ds(g * 16, 16)] >> 9
        pltpu.async_copy(h2_hbm.at[idx], rows, sem)

    def drain(rows, sem):
        pltpu.make_async_copy(h2_hbm.at[pl.ds(0, GSEG2)], rows, sem).wait()

    def rmw(pk, rows):
        for g in range(GSEG2 // 16):
            lv = pk[pl.ds(g * 16, 16)] & 511
            for k in range(16):
                base_w = lv[k] * FS
                for r in range(FS // 16):
                    sl = pl.ds(base_w + r * 16, 16)
                    row = rows[g * 16 + k, pl.ds(r * 16, 16)]
                    acc_v[sl] = jnp.maximum(acc_v[sl], row)

    @pl.when(nch > 0)
    def _():
        prep_fire(0, pkA, idxA, rowsA, semA)

    def pair(j, _):
        c0 = 2 * j
        c1 = c0 + 1

        @pl.when(c1 < nch)
        def _():
            prep_fire(c1, pkB, idxB, rowsB, semB)

        drain(rowsA, semA)
        rmw(pkA, rowsA)

        @pl.when(c0 + 2 < nch)
        def _():
            prep_fire(c0 + 2, pkA, idxA, rowsA, semA)

        @pl.when(c1 < nch)
        def _():
            drain(rowsB, semB)
            rmw(pkB, rowsB)

        return 0

    lax.fori_loop(0, (nch + 1) >> 1, pair, 0)
    pltpu.sync_copy(acc_v.at[pl.ds(0, NPB * FS)],
                    out_hbm.at[pl.ds(lo * FS, NPB * FS)])


def _segmax_call(h2, ptab, cnts):
    f = pl.kernel(
        _segmax_body,
        out_type=jax.ShapeDtypeStruct((NPAD * FS,), jnp.float32),
        mesh=_sc_mesh(),
        scratch_types=[
            pltpu.VMEM((GSEG2,), jnp.int32),
            pltpu.VMEM((GSEG2,), jnp.int32),
            pltpu.VMEM((GSEG2,), jnp.int32),
            pltpu.VMEM((GSEG2,), jnp.int32),
            pltpu.VMEM((GSEG2, FS), jnp.float32),
            pltpu.VMEM((GSEG2, FS), jnp.float32),
            pltpu.VMEM((ACCROWS * FS,), jnp.float32),
            pltpu.VMEM((16,), jnp.int32),
            pltpu.SemaphoreType.DMA,
            pltpu.SemaphoreType.DMA,
        ],
    )
    return f(h2, ptab, cnts).reshape(NPAD, FS)


NCHUNKS = E // GSEG


def _gather_body(g_hbm, src_hbm, dst_hbm, u_hbm, v_hbm,
                 diA, siA, urA, vrA, diB, siB, urB, vrB,
                 gsA, gsB, wsA, wsB):
    w = _wid()
    nl = (NCHUNKS - w + SC_NW - 1) // SC_NW

    def fire(i, di, si, ur, vr, gs):
        base = (w + i * SC_NW) * GSEG
        pltpu.sync_copy(dst_hbm.at[pl.ds(base, GSEG)], di)
        pltpu.sync_copy(src_hbm.at[pl.ds(base, GSEG)], si)
        pltpu.async_copy(g_hbm.at[di], ur, gs)
        pltpu.async_copy(g_hbm.at[si], vr, gs)

    def draing(ur, vr, gs):
        pltpu.make_async_copy(g_hbm.at[pl.ds(0, GSEG)], ur, gs).wait()
        pltpu.make_async_copy(g_hbm.at[pl.ds(0, GSEG)], vr, gs).wait()

    def firewb(i, ur, vr, ws):
        base = (w + i * SC_NW) * GSEG
        pltpu.async_copy(ur, u_hbm.at[pl.ds(base, GSEG)], ws)
        pltpu.async_copy(vr, v_hbm.at[pl.ds(base, GSEG)], ws)

    def drainwb(ur, vr, ws):
        pltpu.make_async_copy(ur, u_hbm.at[pl.ds(0, GSEG)], ws).wait()
        pltpu.make_async_copy(vr, v_hbm.at[pl.ds(0, GSEG)], ws).wait()

    @pl.when(nl > 0)
    def _():
        fire(0, diA, siA, urA, vrA, gsA)

    def pair(j, _):
        c0 = 2 * j
        c1 = c0 + 1

        @pl.when(c1 < nl)
        def _():
            @pl.when(j > 0)
            def _():
                drainwb(urB, vrB, wsB)

            fire(c1, diB, siB, urB, vrB, gsB)

        draing(urA, vrA, gsA)
        firewb(c0, urA, vrA, wsA)

        @pl.when(c0 + 2 < nl)
        def _():
            drainwb(urA, vrA, wsA)
            fire(c0 + 2, diA, siA, urA, vrA, gsA)

        @pl.when(c1 < nl)
        def _():
            draing(urB, vrB, gsB)
            firewb(c1, urB, vrB, wsB)

        return 0

    lax.fori_loop(0, (nl + 1) >> 1, pair, 0)

    @pl.when(nl > 0)
    def _():
        drainwb(urA, vrA, wsA)

    @pl.when(nl > 1)
    def _():
        drainwb(urB, vrB, wsB)


def _gather_call(g, src, dst):
    f = pl.kernel(
        _gather_body,
        out_type=[
            jax.ShapeDtypeStruct((E, FS), jnp.float32),
            jax.ShapeDtypeStruct((E, FS), jnp.float32),
        ],
        mesh=_sc_mesh(),
        scratch_types=[
            pltpu.VMEM((GSEG,), jnp.int32),
            pltpu.VMEM((GSEG,), jnp.int32),
            pltpu.VMEM((GSEG, FS), jnp.float32),
            pltpu.VMEM((GSEG, FS), jnp.float32),
            pltpu.VMEM((GSEG,), jnp.int32),
            pltpu.VMEM((GSEG,), jnp.int32),
            pltpu.VMEM((GSEG, FS), jnp.float32),
            pltpu.VMEM((GSEG, FS), jnp.float32),
            pltpu.SemaphoreType.DMA,
            pltpu.SemaphoreType.DMA,
            pltpu.SemaphoreType.DMA,
            pltpu.SemaphoreType.DMA,
        ],
    )
    return f(g, src, dst)


def _fix_body(a_ref, gp_ref, o_ref):
    a = a_ref[...]
    o_ref[...] = jnp.where(a == NEG_INIT, 0.0, a) + gp_ref[...]


def _fix0_body(a_ref, o_ref):
    a = a_ref[...]
    o_ref[...] = jnp.where(a == NEG_INIT, 0.0, a)


def _fix_call(gacc, gprev):
    nb = 2000
    if gprev is None:
        return pl.pallas_call(
            _fix0_body,
            grid=(N // nb,),
            in_specs=[pl.BlockSpec((nb, FS), lambda i: (i, 0))],
            out_specs=pl.BlockSpec((nb, FS), lambda i: (i, 0)),
            out_shape=jax.ShapeDtypeStruct((N, FS), jnp.float32),
        )(gacc)
    return pl.pallas_call(
        _fix_body,
        grid=(N // nb,),
        in_specs=[
            pl.BlockSpec((nb, FS), lambda i: (i, 0)),
            pl.BlockSpec((nb, FS), lambda i: (i, 0)),
        ],
        out_specs=pl.BlockSpec((nb, FS), lambda i: (i, 0)),
        out_shape=jax.ShapeDtypeStruct((N, FS), jnp.float32),
    )(gacc, gprev)


def _edge_conv(g, src, dst, tabs, gprev, g1, b1, w1, g2, b2, w2):
    u, v = _gather_call(g, src, dst)
    s1 = _stats1_call(u, v)
    h1, s2 = _mm1_call(s1, g1.reshape(2, FS), b1.reshape(2, FS), w1, u, v)
    h2 = _mm2_call(s2, g2.reshape(1, FS), b2.reshape(1, FS), w2, h1)
    ptab, cnts = tabs
    gacc = _segmax_call(h2, ptab, cnts)
    return _fix_call(gacc, gprev)


def kernel(x, edge_index, Wav, bav, l1_g1, l1_b1, l1_W1, l1_g2, l1_b2, l1_W2,
           l2_g1, l2_b1, l2_W1, l2_g2, l2_b2, l2_W2, l3_g1, l3_b1, l3_W1,
           l3_g2, l3_b2, l3_W2, l4_g1, l4_b1, l4_W1, l4_g2, l4_b2, l4_W2,
           Wout, bout):
    src = edge_index[0]
    dst = edge_index[1]
    x2 = x.reshape(N, 2 * D)
    gf = _gf_call(x2, Wav, bav.reshape(1, FS))
    p1 = (l1_g1, l1_b1, l1_W1, l1_g2, l1_b2, l1_W2)
    p2 = (l2_g1, l2_b1, l2_W1, l2_g2, l2_b2, l2_W2)
    p3 = (l3_g1, l3_b1, l3_W1, l3_g2, l3_b2, l3_W2)
    p4 = (l4_g1, l4_b1, l4_W1, l4_g2, l4_b2, l4_W2)
    tabs = _bucket_call(dst)
    g1 = _edge_conv(gf, src, dst, tabs, None, *p1)
    g2 = _edge_conv(g1, src, dst, tabs, g1, *p2)
    g3 = _edge_conv(g2, src, dst, tabs, g2, *p3)
    g4 = _edge_conv(g3, src, dst, tabs, g3, *p4)
    wout_pad = jnp.zeros((FS, FS), jnp.float32).at[:2, :].set(Wout)
    bout_pad = jnp.zeros((1, FS), jnp.float32).at[0, :2].set(bout)
    out_pad = _out_call(g4, wout_pad, bout_pad)
    return out_pad[:, :2]
